# baseline (speedup 1.0000x reference)
import functools

import jax
import jax.numpy as jnp
from jax import lax
from jax.experimental import pallas as pl
from jax.experimental.pallas import tpu as pltpu
from jax.experimental.pallas import tpu_sc as plsc

_V = 16
_D = 4
_NC = 2
_NS = 16
_NW = _NC * _NS


def _table_body(e_ref, h_ref, out_ref):
    t2 = jnp.dot(h_ref[...], e_ref[...].T, preferred_element_type=jnp.float32)
    ki = lax.broadcasted_iota(jnp.int32, (_V, 128), 0)
    ci = lax.broadcasted_iota(jnp.int32, (_V, 128), 1)
    a = (ki == ci % _V).astype(jnp.float32)
    b0 = jnp.dot(t2, a, preferred_element_type=jnp.float32)
    vmask = (ki % 8) == (ci // _V)
    bsel = jnp.where(vmask, b0, 0.0)
    ri = lax.broadcasted_iota(jnp.int32, (2, _V), 0)
    vi = lax.broadcasted_iota(jnp.int32, (2, _V), 1)
    p = (vi // 8 == ri).astype(jnp.float32)
    out_ref[...] = jnp.dot(p, bsel, preferred_element_type=jnp.float32)


def _sc_gather(t_flat, ids4, n_l, n_bt):
    bt_per_w = n_bt // _NW
    mesh = plsc.VectorSubcoreMesh(core_axis_name="c", subcore_axis_name="s")

    @functools.partial(
        pl.kernel,
        out_type=jax.ShapeDtypeStruct((n_l, 2, n_bt, 8, 128), jnp.float32),
        mesh=mesh,
        scratch_types=[
            pltpu.VMEM((2, 128), jnp.float32),
            pltpu.VMEM((bt_per_w, 1, 128), jnp.int32),
            pltpu.VMEM((2, bt_per_w, 8, 128), jnp.float32),
        ],
    )
    def k(tf_hbm, ids_hbm, out_hbm, tcm, idsv, outv):
        w = lax.axis_index("s") * _NC + lax.axis_index("c")
        bt0 = w * bt_per_w
        pltpu.sync_copy(tf_hbm, tcm)
        tvs = [tcm[v // 8, pl.ds((v % 8) * _V, _V)] for v in range(_V)]

        def body(l, carry):
            lt = l // 8
            lr = l % 8
            pltpu.sync_copy(
                ids_hbm.at[lt, pl.ds(bt0, bt_per_w), pl.ds(lr, 1), :], idsv
            )
            for g in range(bt_per_w * 8):
                idv = idsv[g // 8, 0, pl.ds((g % 8) * 16, 16)]
                for v in range(_V):
                    val = lax.gather(
                        tvs[v], idv[:, None],
                        lax.GatherDimensionNumbers(
                            offset_dims=(), collapsed_slice_dims=(0,),
                            start_index_map=(0,)),
                        (1,),
                        mode=lax.GatherScatterMode.PROMISE_IN_BOUNDS)
                    outv[v // 8, g // 8, v % 8, pl.ds((g % 8) * 16, 16)] = val
            pltpu.sync_copy(outv, out_hbm.at[l, :, pl.ds(bt0, bt_per_w), :, :])
            return carry

        lax.fori_loop(0, n_l, body, 0)

    return k(t_flat, ids4)


def kernel(ids, embed_table, head_weight):
    b, l = ids.shape
    n_bt = b // 128
    t_flat = pl.pallas_call(
        _table_body,
        out_shape=jax.ShapeDtypeStruct((2, 128), jnp.float32),
    )(embed_table, head_weight)
    ids4 = ids.T.reshape(l // 8, 8, n_bt, 128).transpose(0, 2, 1, 3)
    out5 = _sc_gather(t_flat, ids4, l, n_bt)
    out_t = out5.transpose(0, 1, 3, 2, 4).reshape(l, _V, b)
    return out_t.transpose(2, 0, 1)


# speedup vs baseline: 1.7168x; 1.7168x over previous
import functools

import jax
import jax.numpy as jnp
from jax import lax
from jax.experimental import pallas as pl
from jax.experimental.pallas import tpu as pltpu
from jax.experimental.pallas import tpu_sc as plsc

_V = 16
_D = 4
_NC = 2
_NS = 16
_NW = _NC * _NS


def _table_body(e_ref, h_ref, out_ref):
    t2 = jnp.dot(h_ref[...], e_ref[...].T, preferred_element_type=jnp.float32)
    ki = lax.broadcasted_iota(jnp.int32, (_V, 128), 0)
    ci = lax.broadcasted_iota(jnp.int32, (_V, 128), 1)
    a = (ki == ci % _V).astype(jnp.float32)
    b0 = jnp.dot(t2, a, preferred_element_type=jnp.float32)
    vmask = (ki % 8) == (ci // _V)
    bsel = jnp.where(vmask, b0, 0.0)
    ri = lax.broadcasted_iota(jnp.int32, (2, _V), 0)
    vi = lax.broadcasted_iota(jnp.int32, (2, _V), 1)
    p = (vi // 8 == ri).astype(jnp.float32)
    out_ref[...] = jnp.dot(p, bsel, preferred_element_type=jnp.float32)


def _sc_gather(t_flat, ids4, n_l, n_bt):
    bt_per_w = n_bt // _NW
    mesh = plsc.VectorSubcoreMesh(core_axis_name="c", subcore_axis_name="s")

    @functools.partial(
        pl.kernel,
        out_type=jax.ShapeDtypeStruct((n_l, 2, n_bt, 8, 128), jnp.float32),
        mesh=mesh,
        scratch_types=[
            pltpu.VMEM((2, 128), jnp.float32),
            pltpu.VMEM((bt_per_w, 1, 128), jnp.int32),
            pltpu.VMEM((bt_per_w, 1, 128), jnp.int32),
            pltpu.VMEM((2, bt_per_w, 8, 128), jnp.float32),
            pltpu.VMEM((2, bt_per_w, 8, 128), jnp.float32),
            pltpu.SemaphoreType.DMA,
            pltpu.SemaphoreType.DMA,
            pltpu.SemaphoreType.DMA,
            pltpu.SemaphoreType.DMA,
        ],
    )
    def k(tf_hbm, ids_hbm, out_hbm, tcm, idsv0, idsv1, outv0, outv1,
          semi0, semi1, semo0, semo1):
        w = lax.axis_index("s") * _NC + lax.axis_index("c")
        bt0 = w * bt_per_w
        pltpu.sync_copy(tf_hbm, tcm)
        tvs = [tcm[v // 8, pl.ds((v % 8) * _V, _V)] for v in range(_V)]

        def fire_ids(l, idsv, semi):
            return pltpu.async_copy(
                ids_hbm.at[l // 8, pl.ds(bt0, bt_per_w), pl.ds(l % 8, 1), :],
                idsv, semi)

        def fire_out(l, outv, semo):
            return pltpu.async_copy(
                outv, out_hbm.at[l, :, pl.ds(bt0, bt_per_w), :, :], semo)

        def wait_ids(idsv, semi):
            pltpu.make_async_copy(
                ids_hbm.at[0, pl.ds(bt0, bt_per_w), pl.ds(0, 1), :],
                idsv, semi).wait()

        def wait_out(outv, semo):
            pltpu.make_async_copy(
                outv, out_hbm.at[0, :, pl.ds(bt0, bt_per_w), :, :],
                semo).wait()

        def compute(idsv, outv):
            for g in range(bt_per_w * 8):
                idv = idsv[g // 8, 0, pl.ds((g % 8) * 16, 16)]
                for v in range(_V):
                    val = lax.gather(
                        tvs[v], idv[:, None],
                        lax.GatherDimensionNumbers(
                            offset_dims=(), collapsed_slice_dims=(0,),
                            start_index_map=(0,)),
                        (1,),
                        mode=lax.GatherScatterMode.PROMISE_IN_BOUNDS)
                    outv[v // 8, g // 8, v % 8, pl.ds((g % 8) * 16, 16)] = val

        fire_ids(0, idsv0, semi0)
        wait_ids(idsv0, semi0)
        fire_ids(1, idsv1, semi1)
        compute(idsv0, outv0)
        fire_out(0, outv0, semo0)
        wait_ids(idsv1, semi1)
        fire_ids(2, idsv0, semi0)
        compute(idsv1, outv1)
        fire_out(1, outv1, semo1)

        def body(i, carry):
            l0 = 2 * i
            wait_ids(idsv0, semi0)
            fire_ids(l0 + 1, idsv1, semi1)
            wait_out(outv0, semo0)
            compute(idsv0, outv0)
            fire_out(l0, outv0, semo0)
            wait_ids(idsv1, semi1)
            nxt = lax.rem(l0 + 2, n_l)
            fire_ids(nxt, idsv0, semi0)
            wait_out(outv1, semo1)
            compute(idsv1, outv1)
            fire_out(l0 + 1, outv1, semo1)
            return carry

        lax.fori_loop(1, n_l // 2, body, 0)
        wait_ids(idsv0, semi0)
        wait_out(outv0, semo0)
        wait_out(outv1, semo1)

    return k(t_flat, ids4)


def kernel(ids, embed_table, head_weight):
    b, l = ids.shape
    n_bt = b // 128
    t_flat = pl.pallas_call(
        _table_body,
        out_shape=jax.ShapeDtypeStruct((2, 128), jnp.float32),
    )(embed_table, head_weight)
    ids4 = ids.T.reshape(l // 8, 8, n_bt, 128).transpose(0, 2, 1, 3)
    out5 = _sc_gather(t_flat, ids4, l, n_bt)
    out_t = out5.transpose(0, 1, 3, 2, 4).reshape(l, _V, b)
    return out_t.transpose(2, 0, 1)


# speedup vs baseline: 1.7299x; 1.0076x over previous
import functools

import jax
import jax.numpy as jnp
from jax import lax
from jax.experimental import pallas as pl
from jax.experimental.pallas import tpu as pltpu
from jax.experimental.pallas import tpu_sc as plsc

_V = 16
_D = 4
_NC = 2
_NS = 16
_NW = _NC * _NS


def _table_body(e_ref, h_ref, out_ref):
    t2 = jnp.dot(h_ref[...], e_ref[...].T, preferred_element_type=jnp.float32)
    ki = lax.broadcasted_iota(jnp.int32, (_V, 128), 0)
    ci = lax.broadcasted_iota(jnp.int32, (_V, 128), 1)
    a = (ki == ci % _V).astype(jnp.float32)
    b0 = jnp.dot(t2, a, preferred_element_type=jnp.float32)
    vmask = (ki % 8) == (ci // _V)
    bsel = jnp.where(vmask, b0, 0.0)
    ri = lax.broadcasted_iota(jnp.int32, (2, _V), 0)
    vi = lax.broadcasted_iota(jnp.int32, (2, _V), 1)
    p = (vi // 8 == ri).astype(jnp.float32)
    out_ref[...] = jnp.dot(p, bsel, preferred_element_type=jnp.float32)


def _sc_gather(t_flat, ids4, n_l, n_bt):
    bt_per_w = n_bt // _NW
    mesh = plsc.VectorSubcoreMesh(core_axis_name="c", subcore_axis_name="s")

    @functools.partial(
        pl.kernel,
        out_type=jax.ShapeDtypeStruct((n_l, 2, n_bt, 8, 128), jnp.float32),
        mesh=mesh,
        scratch_types=[
            pltpu.VMEM((2, 128), jnp.float32),
            pltpu.VMEM((bt_per_w, 1, 128), jnp.int32),
            pltpu.VMEM((bt_per_w, 1, 128), jnp.int32),
            pltpu.VMEM((2, bt_per_w, 8, 128), jnp.float32),
            pltpu.VMEM((2, bt_per_w, 8, 128), jnp.float32),
            pltpu.SemaphoreType.DMA,
            pltpu.SemaphoreType.DMA,
            pltpu.SemaphoreType.DMA,
            pltpu.SemaphoreType.DMA,
        ],
    )
    def k(tf_hbm, ids_hbm, out_hbm, tcm, idsv0, idsv1, outv0, outv1,
          semi0, semi1, semo0, semo1):
        w = lax.axis_index("s") * _NC + lax.axis_index("c")
        bt0 = w * bt_per_w
        pltpu.sync_copy(tf_hbm, tcm)
        tvs = [tcm[v // 8, pl.ds((v % 8) * _V, _V)] for v in range(_V)]

        def fire_ids(l, idsv, semi):
            return pltpu.async_copy(
                ids_hbm.at[l // 8, pl.ds(bt0, bt_per_w), pl.ds(l % 8, 1), :],
                idsv, semi)

        def fire_out(l, outv, semo):
            return pltpu.async_copy(
                outv, out_hbm.at[l, :, pl.ds(bt0, bt_per_w), :, :], semo)

        def wait_ids(idsv, semi):
            pltpu.make_async_copy(
                ids_hbm.at[0, pl.ds(bt0, bt_per_w), pl.ds(0, 1), :],
                idsv, semi).wait()

        def wait_out(outv, semo):
            pltpu.make_async_copy(
                outv, out_hbm.at[0, :, pl.ds(bt0, bt_per_w), :, :],
                semo).wait()

        def compute(idsv, outv):
            @plsc.parallel_loop(0, bt_per_w * 8, unroll=8)
            def _(g):
                gt = g // 8
                go = (g % 8) * 16
                idv = idsv[gt, 0, pl.ds(go, 16)]
                for v in range(_V):
                    val = lax.gather(
                        tvs[v], idv[:, None],
                        lax.GatherDimensionNumbers(
                            offset_dims=(), collapsed_slice_dims=(0,),
                            start_index_map=(0,)),
                        (1,),
                        mode=lax.GatherScatterMode.PROMISE_IN_BOUNDS)
                    outv[v // 8, gt, v % 8, pl.ds(go, 16)] = val

        fire_ids(0, idsv0, semi0)
        wait_ids(idsv0, semi0)
        fire_ids(1, idsv1, semi1)
        compute(idsv0, outv0)
        fire_out(0, outv0, semo0)
        wait_ids(idsv1, semi1)
        fire_ids(2, idsv0, semi0)
        compute(idsv1, outv1)
        fire_out(1, outv1, semo1)

        def body(i, carry):
            l0 = 2 * i
            wait_ids(idsv0, semi0)
            fire_ids(l0 + 1, idsv1, semi1)
            wait_out(outv0, semo0)
            compute(idsv0, outv0)
            fire_out(l0, outv0, semo0)
            wait_ids(idsv1, semi1)
            nxt = lax.rem(l0 + 2, n_l)
            fire_ids(nxt, idsv0, semi0)
            wait_out(outv1, semo1)
            compute(idsv1, outv1)
            fire_out(l0 + 1, outv1, semo1)
            return carry

        lax.fori_loop(1, n_l // 2, body, 0)
        wait_ids(idsv0, semi0)
        wait_out(outv0, semo0)
        wait_out(outv1, semo1)

    return k(t_flat, ids4)


def kernel(ids, embed_table, head_weight):
    b, l = ids.shape
    n_bt = b // 128
    t_flat = pl.pallas_call(
        _table_body,
        out_shape=jax.ShapeDtypeStruct((2, 128), jnp.float32),
    )(embed_table, head_weight)
    ids4 = ids.T.reshape(l // 8, 8, n_bt, 128).transpose(0, 2, 1, 3)
    out5 = _sc_gather(t_flat, ids4, l, n_bt)
    out_t = out5.transpose(0, 1, 3, 2, 4).reshape(l, _V, b)
    return out_t.transpose(2, 0, 1)
